# baseline (speedup 1.0000x reference)
import functools

import jax
import jax.numpy as jnp
from jax import lax
from jax.experimental import pallas as pl
from jax.experimental.pallas import tpu as pltpu
from jax.experimental.pallas import tpu_sc as plsc

D = 128
P = 50
NW = 32
B = 4096
C = 128
NCH = P
NBUF = 5
PREF = 3

_mesh = plsc.VectorSubcoreMesh(core_axis_name="c", subcore_axis_name="s")


@functools.partial(
    pl.kernel,
    out_type=jax.ShapeDtypeStruct((P, B, D), jnp.float32),
    mesh=_mesh,
    scratch_types=[
        pltpu.VMEM((NCH, C), jnp.int32),
        pltpu.VMEM((NCH, C), jnp.int32),
        pltpu.VMEM((NBUF, C, D), jnp.float32),
        pltpu.SemaphoreType.DMA((NBUF,)),
        pltpu.SemaphoreType.DMA((NBUF,)),
        pltpu.SemaphoreType.DMA((NBUF,)),
    ],
    compiler_params=pltpu.CompilerParams(needs_layout_passes=False),
)
def _emb_lookup(xt_hbm, post_hbm, tok_tab_hbm, pos_tab_hbm, out_hbm,
                tok_idx, pos_idx, buf_tok,
                sem_tok, sem_out, sem_pos):
    w = lax.axis_index("s") * 2 + lax.axis_index("c")
    col = w * C
    pltpu.sync_copy(xt_hbm.at[:, pl.ds(col, C)], tok_idx)
    pltpu.sync_copy(post_hbm.at[:, pl.ds(col, C)], pos_idx)

    def start_gather(c, b):
        pltpu.async_copy(tok_tab_hbm.at[tok_idx.at[c]], buf_tok.at[b],
                         sem_tok.at[b])

    def wait_gather(b):
        pltpu.make_async_copy(tok_tab_hbm.at[tok_idx.at[0]], buf_tok.at[b],
                              sem_tok.at[b]).wait()

    def wait_out(b):
        pltpu.make_async_copy(buf_tok.at[b], out_hbm.at[0, pl.ds(col, C)],
                              sem_out.at[b]).wait()

    def add_and_store(c, b):
        pltpu.async_copy(pos_tab_hbm.at[pos_idx.at[c]], buf_tok.at[b],
                         sem_pos.at[b], add=True)
        pltpu.make_async_copy(pos_tab_hbm.at[pos_idx.at[0]], buf_tok.at[b],
                              sem_pos.at[b]).wait()
        pltpu.async_copy(buf_tok.at[b], out_hbm.at[c, pl.ds(col, C)],
                         sem_out.at[b])

    def step(c, b, prefetch, first_use):
        bp = (b + PREF) % NBUF
        if prefetch:
            if not first_use:
                wait_out(bp)
            start_gather(c + PREF, bp)
        wait_gather(b)
        add_and_store(c, b)

    for b in range(PREF):
        start_gather(b, b)
    for b in range(NBUF):
        step(b, b, prefetch=True, first_use=(b + PREF < NBUF))

    def group(g, carry):
        for b in range(NBUF):
            step(g * NBUF + b, b, prefetch=True, first_use=False)
        return carry

    lax.fori_loop(1, NCH // NBUF - 1, group, 0)

    for b in range(NBUF):
        c = NCH - NBUF + b
        step(c, b, prefetch=(c + PREF < NCH), first_use=False)
    for b in range(NBUF):
        wait_out(b)


def kernel(x, pos, token_embed, pos_embed):
    out = _emb_lookup(x.T, pos.T, token_embed, pos_embed)
    return out.transpose(1, 0, 2)


# speedup vs baseline: 2.4041x; 2.4041x over previous
import functools

import jax
import jax.numpy as jnp
from jax import lax
from jax.experimental import pallas as pl
from jax.experimental.pallas import tpu as pltpu
from jax.experimental.pallas import tpu_sc as plsc

D = 128
P = 50
NW = 32
B = 4096
C = 128
NCH = P
NBUF = 5
PREF = 3

_mesh = plsc.VectorSubcoreMesh(core_axis_name="c", subcore_axis_name="s")


@functools.partial(
    pl.kernel,
    out_type=jax.ShapeDtypeStruct((P, B, D), jnp.float32),
    mesh=_mesh,
    scratch_types=[
        pltpu.VMEM((NCH, C), jnp.int32),
        pltpu.VMEM((NCH, C), jnp.int32),
        pltpu.VMEM((NBUF, C, D), jnp.float32),
        pltpu.SemaphoreType.DMA((NBUF,)),
        pltpu.SemaphoreType.DMA((NBUF,)),
    ],
    compiler_params=pltpu.CompilerParams(needs_layout_passes=False),
)
def _emb_lookup(xt_hbm, post_hbm, tok_tab_hbm, pos_tab_hbm, out_hbm,
                tok_idx, pos_idx, buf_tok,
                sem_tok, sem_out):
    pl.run_scoped(
        functools.partial(_emb_lookup_body, xt_hbm, post_hbm, tok_tab_hbm,
                          pos_tab_hbm, out_hbm, tok_idx, pos_idx, buf_tok,
                          sem_tok, sem_out),
        pltpu.VMEM((P, D), jnp.float32),
    )


def _emb_lookup_body(xt_hbm, post_hbm, tok_tab_hbm, pos_tab_hbm, out_hbm,
                     tok_idx, pos_idx, buf_tok, sem_tok, sem_out, pos_tab):
    w = lax.axis_index("s") * 2 + lax.axis_index("c")
    col = w * C
    pltpu.sync_copy(xt_hbm.at[:, pl.ds(col, C)], tok_idx)
    pltpu.sync_copy(post_hbm.at[:, pl.ds(col, C)], pos_idx)
    pltpu.sync_copy(pos_tab_hbm, pos_tab)

    lanes = jnp.arange(16, dtype=jnp.int32)

    def start_gather(c, b):
        pltpu.async_copy(tok_tab_hbm.at[tok_idx.at[c]], buf_tok.at[b],
                         sem_tok.at[b])

    def wait_gather(b):
        pltpu.make_async_copy(tok_tab_hbm.at[tok_idx.at[0]], buf_tok.at[b],
                              sem_tok.at[b]).wait()

    def wait_out(b):
        pltpu.make_async_copy(buf_tok.at[b], out_hbm.at[0, pl.ds(col, C)],
                              sem_out.at[b]).wait()

    lane_consts = [jnp.full((16,), r, dtype=jnp.int32) for r in range(16)]

    def add_and_store(c, b):
        def add_group(g, carry2):
            pvec = pos_idx[c, pl.ds(g * 16, 16)]
            i0 = g * 16
            for r in range(16):
                prow = pvec.at[lane_consts[r]].get(mode="promise_in_bounds")
                for j in range(D // 16):
                    vals = plsc.load_gather(pos_tab, [prow, lanes + (j * 16)])
                    plsc.addupdate(buf_tok.at[b, i0 + r, pl.ds(j * 16, 16)],
                                   vals)
            return carry2

        lax.fori_loop(0, C // 16, add_group, 0)
        pltpu.async_copy(buf_tok.at[b], out_hbm.at[c, pl.ds(col, C)],
                         sem_out.at[b])

    def step(c, b, prefetch, first_use):
        bp = (b + PREF) % NBUF
        if prefetch:
            if not first_use:
                wait_out(bp)
            start_gather(c + PREF, bp)
        wait_gather(b)
        add_and_store(c, b)

    for b in range(PREF):
        start_gather(b, b)
    for b in range(NBUF):
        step(b, b, prefetch=True, first_use=(b + PREF < NBUF))

    def group(g, carry):
        for b in range(NBUF):
            step(g * NBUF + b, b, prefetch=True, first_use=False)
        return carry

    lax.fori_loop(1, NCH // NBUF - 1, group, 0)

    for b in range(NBUF):
        c = NCH - NBUF + b
        step(c, b, prefetch=(c + PREF < NCH), first_use=False)
    for b in range(NBUF):
        wait_out(b)


def kernel(x, pos, token_embed, pos_embed):
    out = _emb_lookup(x.T, pos.T, token_embed, pos_embed)
    return out.transpose(1, 0, 2)
